# baseline (speedup 1.0000x reference)
import functools

import jax
import jax.numpy as jnp
from jax import lax
from jax.experimental import pallas as pl
from jax.experimental.pallas import tpu as pltpu
from jax.experimental.pallas import tpu_sc as plsc

_N = 1000
_B = 16384
_RB = 200
_NRB = _N // _RB
_NC = 2
_NS = 16
_LANES = 16
_NW = _NC * _NS
_BPW = _B // _NW
_CHUNKS = _BPW // _LANES
_ROWS = _BPW // 128


def _tab_body(wa_ref, wba_ref, wpc_ref, wa_col_ref, lse_a_ref):
    @pl.when(pl.program_id(0) == 0)
    def _():
        wa = wa_ref[...]
        m_a = jnp.max(wa)
        s_a = jnp.sum(jnp.exp(wa - m_a))
        lse_a_ref[0, 0] = jnp.log(s_a) + m_a
        wa_col_ref[...] = wa.reshape(_N, 1)

    r = pl.program_id(0)
    wba = wba_ref[...]
    m_b = jnp.max(wba, axis=1, keepdims=True)
    s_b = jnp.sum(jnp.exp(wba - m_b), axis=1, keepdims=True)
    lse_b = jnp.log(s_b) + m_b
    wa_col = wa_col_ref[pl.ds(r * _RB, _RB), :]
    g_col = wa_col - (lse_b + lse_a_ref[0, 0])
    for k in range(7):
        wpc_ref[:, k, :] = wba[:, k * 128:(k + 1) * 128] + g_col
    wpc_ref[:, 7, : _N - 7 * 128] = wba[:, 7 * 128:] + g_col


def _sc_body(wpc_hbm, ia_hbm, ib_hbm, out_hbm, ia_v, ib_v, f_v, wv, sem):
    wid = lax.axis_index("s") * _NC + lax.axis_index("c")
    base = wid * _BPW
    pltpu.sync_copy(ia_hbm.at[pl.ds(base, _BPW)], ia_v)
    pltpu.sync_copy(ib_hbm.at[pl.ds(base, _BPW)], ib_v)

    def chunk(j):
        a16 = ia_v[pl.ds(j * _LANES, _LANES)]
        c16 = ib_v[pl.ds(j * _LANES, _LANES)]
        f_v[pl.ds(j * _LANES, _LANES)] = a16 * 1024 + c16

    pl.loop(0, _CHUNKS)(chunk)
    copies = []
    for r in range(_ROWS):
        copies.append(
            pltpu.async_copy(wpc_hbm.at[f_v.at[pl.ds(r * 128, 128)]],
                             wv.at[pl.ds(r * 128, 128)], sem))
    for cp in copies:
        cp.wait()
    pltpu.sync_copy(wv, out_hbm.at[pl.ds(base, _BPW)])


@functools.cache
def _sc_gather():
  return pl.kernel(
    _sc_body,
    out_type=jax.ShapeDtypeStruct((_B,), jnp.float32),
    mesh=plsc.VectorSubcoreMesh(core_axis_name="c", subcore_axis_name="s",
                                num_cores=_NC, num_subcores=_NS),
    scratch_types=[
        pltpu.VMEM((_BPW,), jnp.int32),
        pltpu.VMEM((_BPW,), jnp.int32),
        pltpu.VMEM((_BPW,), jnp.int32),
        pltpu.VMEM((_BPW,), jnp.float32),
        pltpu.SemaphoreType.DMA,
    ],
  )


def kernel(inputs, w_A, w_BA):
    idx = inputs.astype(jnp.int32)
    idx_a = idx[:, 0]
    idx_b = idx[:, 1]
    wpc = pl.pallas_call(
        _tab_body,
        grid=(_NRB,),
        in_specs=[
            pl.BlockSpec((_N,), lambda r: (0,)),
            pl.BlockSpec((_RB, _N), lambda r: (r, 0)),
        ],
        out_specs=pl.BlockSpec((_RB, 8, 128), lambda r: (r, 0, 0)),
        out_shape=jax.ShapeDtypeStruct((_N, 8, 128), jnp.float32),
        scratch_shapes=[
            pltpu.VMEM((_N, 1), jnp.float32),
            pltpu.SMEM((1, 1), jnp.float32),
        ],
    )(w_A, w_BA)
    return _sc_gather()(wpc.reshape(-1), idx_a, idx_b)


# speedup vs baseline: 1.0478x; 1.0478x over previous
import functools

import jax
import jax.numpy as jnp
from jax import lax
from jax.experimental import pallas as pl
from jax.experimental.pallas import tpu as pltpu
from jax.experimental.pallas import tpu_sc as plsc

_N = 1000
_B = 16384
_NC = 2
_NS = 16
_NW = _NC * _NS
_BPW = _B // _NW
_ROWS = _BPW // 128


def _tab_body(wa_ref, wba_ref, wpc_ref):
    wa = wa_ref[...]
    m_a = jnp.max(wa)
    s_a = jnp.sum(jnp.exp(wa - m_a))
    lse_a = jnp.log(s_a) + m_a
    wba = wba_ref[...]
    m_b = jnp.max(wba, axis=1, keepdims=True)
    s_b = jnp.sum(jnp.exp(wba - m_b), axis=1, keepdims=True)
    lse_b = jnp.log(s_b) + m_b
    g_col = wa.reshape(_N, 1) - (lse_b + lse_a)
    for k in range(7):
        wpc_ref[:, k, :] = wba[:, k * 128:(k + 1) * 128] + g_col
    wpc_ref[:, 7, : _N - 7 * 128] = wba[:, 7 * 128:] + g_col


def _sc_body(wpc_hbm, f_hbm, out_hbm, f_v, wv, sem):
    wid = lax.axis_index("s") * _NC + lax.axis_index("c")
    base = wid * _BPW
    pltpu.sync_copy(f_hbm.at[pl.ds(base, _BPW)], f_v)
    copies = []
    for r in range(_ROWS):
        copies.append(
            pltpu.async_copy(wpc_hbm.at[f_v.at[pl.ds(r * 128, 128)]],
                             wv.at[pl.ds(r * 128, 128)], sem))
    for cp in copies:
        cp.wait()
    pltpu.sync_copy(wv, out_hbm.at[pl.ds(base, _BPW)])


@functools.cache
def _sc_gather():
  return pl.kernel(
    _sc_body,
    out_type=jax.ShapeDtypeStruct((_B,), jnp.float32),
    mesh=plsc.VectorSubcoreMesh(core_axis_name="c", subcore_axis_name="s",
                                num_cores=_NC, num_subcores=_NS),
    scratch_types=[
        pltpu.VMEM((_BPW,), jnp.int32),
        pltpu.VMEM((_BPW,), jnp.float32),
        pltpu.SemaphoreType.DMA,
    ],
  )


def kernel(inputs, w_A, w_BA):
    idx = inputs.astype(jnp.int32)
    f = idx[:, 0] * 1024 + idx[:, 1]
    wpc = pl.pallas_call(
        _tab_body,
        out_shape=jax.ShapeDtypeStruct((_N, 8, 128), jnp.float32),
    )(w_A, w_BA)
    return _sc_gather()(wpc.reshape(-1), f)
